# baseline (speedup 1.0000x reference)
import functools

import jax
import jax.numpy as jnp
from jax import lax
from jax.experimental import pallas as pl
from jax.experimental.pallas import tpu as pltpu
from jax.experimental.pallas import tpu_sc as plsc

_K = 128
_NTILES = 32
_GROWS = 16


def _mesh():
    return plsc.VectorSubcoreMesh(core_axis_name="c", subcore_axis_name="s")


def _sc_deg(dstp2d, n_pad):
    nrow_d = dstp2d.shape[0]
    per_tile = nrow_d // 32
    rows_t = n_pad // 16
    orows_t = n_pad // 128

    @functools.partial(
        pl.kernel,
        out_type=jax.ShapeDtypeStruct((2, n_pad // 8, 128), jnp.float32),
        mesh=_mesh(),
        scratch_types=[
            pltpu.VMEM_SHARED((n_pad, 16), jnp.float32),
            pltpu.VMEM((2, _K), jnp.int32),
            pltpu.VMEM((_K, 16), jnp.float32),
            pltpu.VMEM((n_pad // 16, 16), jnp.float32),
            pltpu.VMEM((n_pad // 128, 128), jnp.float32),
            pltpu.SemaphoreType.DMA,
        ],
    )
    def deg_kernel(dst_hbm, out_hbm, acc, idxd, ones_v, pbuf, obuf, sem_i):
        c = lax.axis_index("c")
        s = lax.axis_index("s")
        one16 = jnp.full((16,), 1.0, jnp.float32)
        zero16 = jnp.zeros((16,), jnp.float32)

        def fill_ones(j, carry):
            ones_v[j] = one16
            return carry

        lax.fori_loop(0, _K, fill_ones, 0)

        def fill_zero(j, carry):
            pbuf[j] = zero16
            return carry

        lax.fori_loop(0, rows_t, fill_zero, 0)
        pltpu.sync_copy(pbuf, acc.at[pl.ds(s * rows_t, rows_t)])
        base = (c * 16 + s) * per_tile
        pltpu.sync_copy(dst_hbm.at[base], idxd.at[0])
        plsc.subcore_barrier()
        pltpu.async_copy(dst_hbm.at[base + 1], idxd.at[1], sem_i)

        def body(jj, carry):
            for b in range(2):
                j = jj * 2 + b
                nb = 1 - b
                pltpu.make_async_copy(
                    dst_hbm.at[base], idxd.at[nb], sem_i).wait()
                pltpu.sync_copy(ones_v, acc.at[idxd.at[b]], add=True)
                jn = lax.rem(j + 2, per_tile)
                pltpu.async_copy(dst_hbm.at[base + jn], idxd.at[b], sem_i)
            return carry

        lax.fori_loop(0, per_tile // 2, body, 0)
        pltpu.make_async_copy(dst_hbm.at[base], idxd.at[0], sem_i).wait()
        plsc.subcore_barrier()
        pltpu.sync_copy(acc.at[pl.ds(s * rows_t, rows_t)], pbuf)

        def pack(j, carry):
            obuf[lax.div(j, 8), pl.ds(lax.rem(j, 8) * 16, 16)] = pbuf[j]
            return carry

        lax.fori_loop(0, rows_t, pack, 0)
        pltpu.sync_copy(obuf, out_hbm.at[c, pl.ds(s * orows_t, orows_t)])

    return deg_kernel(dstp2d)


def _sc_agg(y_packed, src2d, dstp2d, zeros_wide, n_pad, hh, core_split):
    nrow_d = dstp2d.shape[0]
    per_tile = nrow_d // (32 if core_split else 16)
    rows_t = n_pad // 16

    @functools.partial(
        pl.kernel,
        out_type=jax.ShapeDtypeStruct((2, n_pad, hh), jnp.float32),
        mesh=_mesh(),
        scratch_types=[
            pltpu.VMEM_SHARED((n_pad, hh), jnp.float32),
            pltpu.VMEM((2, _K), jnp.int32),
            pltpu.VMEM((2, _K), jnp.int32),
            pltpu.VMEM((2, _K, hh), jnp.float32),
            pltpu.SemaphoreType.DMA,
            pltpu.SemaphoreType.DMA,
        ],
    )
    def agg_kernel(y_hbm, src_hbm, dst_hbm, zeros_hbm, out_hbm,
                   acc, idxs, idxd, rows_v, sem_g, sem_i):
        c = lax.axis_index("c")
        s = lax.axis_index("s")
        pltpu.sync_copy(
            zeros_hbm.at[pl.ds(s * rows_t, rows_t)],
            acc.at[pl.ds(s * rows_t, rows_t)],
        )
        if core_split:
            base_s = (c * 16 + s) * per_tile
            base_d = base_s
        else:
            base_s = c * nrow_d + s * per_tile
            base_d = s * per_tile
        pltpu.sync_copy(src_hbm.at[base_s], idxs.at[0])
        pltpu.sync_copy(dst_hbm.at[base_d], idxd.at[0])
        plsc.subcore_barrier()
        pltpu.async_copy(y_hbm.at[idxs.at[0]], rows_v.at[0], sem_g)
        pltpu.async_copy(src_hbm.at[base_s + 1], idxs.at[1], sem_i)
        pltpu.async_copy(dst_hbm.at[base_d + 1], idxd.at[1], sem_i)

        def body(jj, carry):
            for b in range(2):
                j = jj * 2 + b
                nb = 1 - b
                pltpu.make_async_copy(
                    src_hbm.at[base_s], idxs.at[nb], sem_i).wait()
                pltpu.make_async_copy(
                    dst_hbm.at[base_d], idxd.at[nb], sem_i).wait()
                pltpu.make_async_copy(
                    y_hbm.at[idxs.at[b]], rows_v.at[b], sem_g).wait()
                pltpu.async_copy(
                    y_hbm.at[idxs.at[nb]], rows_v.at[nb], sem_g)
                pltpu.sync_copy(rows_v.at[b], acc.at[idxd.at[b]], add=True)
                jn = lax.rem(j + 2, per_tile)
                pltpu.async_copy(src_hbm.at[base_s + jn], idxs.at[b], sem_i)
                pltpu.async_copy(dst_hbm.at[base_d + jn], idxd.at[b], sem_i)
            return carry

        lax.fori_loop(0, per_tile // 2, body, 0)
        pltpu.make_async_copy(y_hbm.at[idxs.at[0]], rows_v.at[0], sem_g).wait()
        pltpu.make_async_copy(src_hbm.at[base_s], idxs.at[0], sem_i).wait()
        pltpu.make_async_copy(dst_hbm.at[base_d], idxd.at[0], sem_i).wait()
        plsc.subcore_barrier()
        pltpu.sync_copy(
            acc.at[pl.ds(s * rows_t, rows_t)],
            out_hbm.at[c, pl.ds(s * rows_t, rows_t)],
        )

    return agg_kernel(y_packed, src2d, dstp2d, zeros_wide)


def _sc_agg2(y2, src2d, dstp2d, n_pad, hh):
    nrow_d = dstp2d.shape[0]
    per_tile = nrow_d // 32
    rows_t = n_pad // 16
    orows_t = n_pad // 128

    @functools.partial(
        pl.kernel,
        out_type=jax.ShapeDtypeStruct((2, n_pad // 8, 128), jnp.float32),
        mesh=_mesh(),
        scratch_types=[
            pltpu.VMEM_SHARED((n_pad, 16), jnp.float32),
            pltpu.VMEM((2, _K), jnp.int32),
            pltpu.VMEM((2, _K), jnp.int32),
            pltpu.VMEM((_K, hh), jnp.float32),
            pltpu.VMEM((_K, 16), jnp.float32),
            pltpu.VMEM((n_pad // 16, 16), jnp.float32),
            pltpu.SemaphoreType.DMA,
            pltpu.SemaphoreType.DMA,
        ],
    )
    def agg2_kernel(y_hbm, src_hbm, dst_hbm, out_hbm,
                    acc, idxs, idxd, rows_v, rows16, pbuf,
                    sem_g, sem_i):
        c = lax.axis_index("c")
        s = lax.axis_index("s")
        zero16 = jnp.zeros((16,), jnp.float32)

        def fill_zero(j, carry):
            pbuf[j] = zero16
            return carry

        lax.fori_loop(0, rows_t, fill_zero, 0)
        pltpu.sync_copy(pbuf, acc.at[pl.ds(s * rows_t, rows_t)])
        base = (c * 16 + s) * per_tile
        pltpu.sync_copy(src_hbm.at[base], idxs.at[0])
        pltpu.sync_copy(dst_hbm.at[base], idxd.at[0])
        plsc.subcore_barrier()
        pltpu.async_copy(y_hbm.at[idxs.at[0]], rows_v, sem_g)
        pltpu.async_copy(src_hbm.at[base + 1], idxs.at[1], sem_i)
        pltpu.async_copy(dst_hbm.at[base + 1], idxd.at[1], sem_i)

        def body(jj, carry):
            for b in range(2):
                j = jj * 2 + b
                nb = 1 - b
                pltpu.make_async_copy(
                    src_hbm.at[base], idxs.at[nb], sem_i).wait()
                pltpu.make_async_copy(
                    dst_hbm.at[base], idxd.at[nb], sem_i).wait()
                pltpu.make_async_copy(
                    y_hbm.at[idxs.at[b]], rows_v, sem_g).wait()

                def compact(i, carry2):
                    rows16[i] = rows_v[i, pl.ds(0, 16)]
                    return carry2

                lax.fori_loop(0, _K, compact, 0)
                pltpu.async_copy(
                    y_hbm.at[idxs.at[nb]], rows_v, sem_g)
                pltpu.sync_copy(rows16, acc.at[idxd.at[b]], add=True)
                jn = lax.rem(j + 2, per_tile)
                pltpu.async_copy(src_hbm.at[base + jn], idxs.at[b], sem_i)
                pltpu.async_copy(dst_hbm.at[base + jn], idxd.at[b], sem_i)
            return carry

        lax.fori_loop(0, per_tile // 2, body, 0)
        pltpu.make_async_copy(y_hbm.at[idxs.at[0]], rows_v, sem_g).wait()
        pltpu.make_async_copy(src_hbm.at[base], idxs.at[0], sem_i).wait()
        pltpu.make_async_copy(dst_hbm.at[base], idxd.at[0], sem_i).wait()
        plsc.subcore_barrier()
        pltpu.sync_copy(acc.at[pl.ds(s * rows_t, rows_t)], pbuf)

        def pack(j, carry):
            rows_v[lax.div(j, 8), pl.ds(lax.rem(j, 8) * 16, 16)] = pbuf[j]
            return carry

        lax.fori_loop(0, rows_t, pack, 0)
        pltpu.sync_copy(rows_v.at[pl.ds(0, orows_t)],
                        out_hbm.at[c, pl.ds(s * orows_t, orows_t)])

    return agg2_kernel(y2, src2d, dstp2d)


def _dinv_of(degp_blk):
    deg = degp_blk[0, :, 0] + degp_blk[1, :, 0] + 1.0
    return lax.rsqrt(deg)


def _tc_prep1(x, W1, degp, r):
    n, d = x.shape
    h = W1.shape[1]
    hh = h // 2
    grid = (n // r,)

    def body(x_ref, w_ref, degp_ref, y1_ref):
        dinv = _dinv_of(degp_ref)
        xw = jnp.dot(x_ref[...], w_ref[...],
                     preferred_element_type=jnp.float32)
        y = xw * dinv[:, None]
        y1_ref[0] = y[:, :hh]
        y1_ref[1] = y[:, hh:]

    return pl.pallas_call(
        body,
        grid=grid,
        in_specs=[
            pl.BlockSpec((r, d), lambda i: (i, 0)),
            pl.BlockSpec((d, h), lambda i: (0, 0)),
            pl.BlockSpec((2, r, 16), lambda i: (0, i, 0)),
        ],
        out_specs=pl.BlockSpec((2, r, hh), lambda i: (0, i, 0)),
        out_shape=jax.ShapeDtypeStruct((2, n, hh), jnp.float32),
    )(x, W1, degp)


def _tc_prep2(agg1, y1, degp, b1, W2p, r):
    _, n, hh = agg1.shape
    h = 2 * hh
    cp = W2p.shape[1]
    grid = (n // r,)

    def body(agg_ref, y1_ref, degp_ref, b1_ref, w2_ref, y2_ref):
        dinv = _dinv_of(degp_ref)
        agg = jnp.concatenate([agg_ref[0], agg_ref[1]], axis=-1)
        y1b = jnp.concatenate([y1_ref[0], y1_ref[1]], axis=-1)
        hact = jnp.maximum(dinv[:, None] * (agg + y1b) + b1_ref[...], 0.0)
        y2 = jnp.dot(hact, w2_ref[...],
                     preferred_element_type=jnp.float32)
        y2_ref[...] = y2 * dinv[:, None]

    return pl.pallas_call(
        body,
        grid=grid,
        in_specs=[
            pl.BlockSpec((2, r, hh), lambda i: (0, i, 0)),
            pl.BlockSpec((2, r, hh), lambda i: (0, i, 0)),
            pl.BlockSpec((2, r, 16), lambda i: (0, i, 0)),
            pl.BlockSpec((1, h), lambda i: (0, 0)),
            pl.BlockSpec((h, cp), lambda i: (0, 0)),
        ],
        out_specs=pl.BlockSpec((r, cp), lambda i: (i, 0)),
        out_shape=jax.ShapeDtypeStruct((n, cp), jnp.float32),
    )(agg1, y1, degp, b1, W2p)


def _tc_final(agg2, y2, degp, b2p, r):
    w = y2.shape[1]
    n = y2.shape[0]
    grid = (n // r,)

    def body(agg_ref, y2_ref, degp_ref, b2_ref, o_ref):
        dinv = _dinv_of(degp_ref)
        o_ref[...] = (dinv[:, None]
                      * (agg_ref[0] + agg_ref[1] + y2_ref[:, :16])
                      + b2_ref[...])

    return pl.pallas_call(
        body,
        grid=grid,
        in_specs=[
            pl.BlockSpec((2, r, 16), lambda i: (0, i, 0)),
            pl.BlockSpec((r, w), lambda i: (i, 0)),
            pl.BlockSpec((2, r, 16), lambda i: (0, i, 0)),
            pl.BlockSpec((1, 16), lambda i: (0, 0)),
        ],
        out_specs=pl.BlockSpec((r, 16), lambda i: (i, 0)),
        out_shape=jax.ShapeDtypeStruct((n, 16), jnp.float32),
    )(agg2, y2, degp, b2p)


def kernel(x, edge_index, W1, b1, W2, b2):
    n, d = x.shape
    h = W1.shape[1]
    c = W2.shape[1]
    e = edge_index.shape[1]
    hh = h // 2

    blk = _K * _NTILES
    ep = ((e + blk - 1) // blk) * blk
    p = ep - e
    pad = jnp.arange(p, dtype=jnp.int32) % _GROWS
    srcp = jnp.concatenate([edge_index[0], pad])
    dstp = jnp.concatenate([edge_index[1], pad + n])
    srcpp = jnp.concatenate([srcp, srcp + n])

    n_pad = ((n + _GROWS + 1023) // 1024) * 1024
    zeros_wide = jnp.zeros((n_pad, hh), jnp.float32)
    W2p = jnp.zeros((h, hh), jnp.float32).at[:, :c].set(W2)
    b1r = b1.reshape(1, h)
    b2p = jnp.zeros((1, 16), jnp.float32).at[0, :c].set(b2)

    r = 2000

    srcp2d = srcp.reshape(-1, _K)
    dstp2d = dstp.reshape(-1, _K)
    srcpp2d = srcpp.reshape(-1, _K)

    degp = _sc_deg(dstp2d, n_pad).reshape(2, n_pad, 16)
    y1 = _tc_prep1(x, W1, degp, r)
    y1_packed = y1.reshape(2 * n, hh)
    agg1 = _sc_agg(y1_packed, srcpp2d, dstp2d, zeros_wide, n_pad, hh, False)
    y2 = _tc_prep2(agg1, y1, degp, b1r, W2p, r)
    agg2 = _sc_agg2(y2, srcp2d, dstp2d, n_pad, hh).reshape(2, n_pad, 16)
    out = _tc_final(agg2, y2, degp, b2p, r)
    return out[:n, :c]


# speedup vs baseline: 1.0752x; 1.0752x over previous
import functools

import jax
import jax.numpy as jnp
from jax import lax
from jax.experimental import pallas as pl
from jax.experimental.pallas import tpu as pltpu
from jax.experimental.pallas import tpu_sc as plsc

_K = 128
_NTILES = 32
_GROWS = 16


def _mesh():
    return plsc.VectorSubcoreMesh(core_axis_name="c", subcore_axis_name="s")


def _sc_deg(dstp2d, n_pad):
    nrow_d = dstp2d.shape[0]
    per_tile = nrow_d // 32
    rows_t = n_pad // 16
    orows_t = n_pad // 128

    @functools.partial(
        pl.kernel,
        out_type=jax.ShapeDtypeStruct((2, n_pad // 8, 128), jnp.float32),
        mesh=_mesh(),
        scratch_types=[
            pltpu.VMEM_SHARED((n_pad, 16), jnp.float32),
            pltpu.VMEM((2, _K), jnp.int32),
            pltpu.VMEM((_K, 16), jnp.float32),
            pltpu.VMEM((n_pad // 16, 16), jnp.float32),
            pltpu.VMEM((n_pad // 128, 128), jnp.float32),
            pltpu.SemaphoreType.DMA,
        ],
    )
    def deg_kernel(dst_hbm, out_hbm, acc, idxd, ones_v, pbuf, obuf, sem_i):
        c = lax.axis_index("c")
        s = lax.axis_index("s")
        one16 = jnp.full((16,), 1.0, jnp.float32)
        zero16 = jnp.zeros((16,), jnp.float32)

        def fill_ones(j, carry):
            ones_v[j] = one16
            return carry

        lax.fori_loop(0, _K, fill_ones, 0)

        def fill_zero(j, carry):
            pbuf[j] = zero16
            return carry

        lax.fori_loop(0, rows_t, fill_zero, 0)
        pltpu.sync_copy(pbuf, acc.at[pl.ds(s * rows_t, rows_t)])
        base = (c * 16 + s) * per_tile
        pltpu.sync_copy(dst_hbm.at[base], idxd.at[0])
        plsc.subcore_barrier()
        pltpu.async_copy(dst_hbm.at[base + 1], idxd.at[1], sem_i)

        def body(jj, carry):
            for b in range(2):
                j = jj * 2 + b
                nb = 1 - b
                pltpu.make_async_copy(
                    dst_hbm.at[base], idxd.at[nb], sem_i).wait()
                pltpu.sync_copy(ones_v, acc.at[idxd.at[b]], add=True)
                jn = lax.rem(j + 2, per_tile)
                pltpu.async_copy(dst_hbm.at[base + jn], idxd.at[b], sem_i)
            return carry

        lax.fori_loop(0, per_tile // 2, body, 0)
        pltpu.make_async_copy(dst_hbm.at[base], idxd.at[0], sem_i).wait()
        plsc.subcore_barrier()
        pltpu.sync_copy(acc.at[pl.ds(s * rows_t, rows_t)], pbuf)

        def pack(j, carry):
            obuf[lax.div(j, 8), pl.ds(lax.rem(j, 8) * 16, 16)] = pbuf[j]
            return carry

        lax.fori_loop(0, rows_t, pack, 0)
        pltpu.sync_copy(obuf, out_hbm.at[c, pl.ds(s * orows_t, orows_t)])

    return deg_kernel(dstp2d)


def _sc_agg(y_packed, src2d, dstp2d, zeros_wide, n_pad, hh, core_split):
    nrow_d = dstp2d.shape[0]
    per_tile = nrow_d // (32 if core_split else 16)
    rows_t = n_pad // 16

    @functools.partial(
        pl.kernel,
        out_type=jax.ShapeDtypeStruct((2, n_pad, hh), jnp.float32),
        mesh=_mesh(),
        scratch_types=[
            pltpu.VMEM_SHARED((n_pad, hh), jnp.float32),
            pltpu.VMEM((2, _K), jnp.int32),
            pltpu.VMEM((2, _K), jnp.int32),
            pltpu.VMEM((2, _K, hh), jnp.float32),
            pltpu.SemaphoreType.DMA,
            pltpu.SemaphoreType.DMA,
        ],
    )
    def agg_kernel(y_hbm, src_hbm, dst_hbm, zeros_hbm, out_hbm,
                   acc, idxs, idxd, rows_v, sem_g, sem_i):
        c = lax.axis_index("c")
        s = lax.axis_index("s")
        pltpu.sync_copy(
            zeros_hbm.at[pl.ds(s * rows_t, rows_t)],
            acc.at[pl.ds(s * rows_t, rows_t)],
        )
        if core_split:
            base_s = (c * 16 + s) * per_tile
            base_d = base_s
        else:
            base_s = c * nrow_d + s * per_tile
            base_d = s * per_tile
        pltpu.sync_copy(src_hbm.at[base_s], idxs.at[0])
        pltpu.sync_copy(dst_hbm.at[base_d], idxd.at[0])
        plsc.subcore_barrier()
        pltpu.async_copy(y_hbm.at[idxs.at[0]], rows_v.at[0], sem_g)
        pltpu.async_copy(src_hbm.at[base_s + 1], idxs.at[1], sem_i)
        pltpu.async_copy(dst_hbm.at[base_d + 1], idxd.at[1], sem_i)

        def body(jj, carry):
            for b in range(2):
                j = jj * 2 + b
                nb = 1 - b
                pltpu.make_async_copy(
                    src_hbm.at[base_s], idxs.at[nb], sem_i).wait()
                pltpu.make_async_copy(
                    dst_hbm.at[base_d], idxd.at[nb], sem_i).wait()
                pltpu.make_async_copy(
                    y_hbm.at[idxs.at[b]], rows_v.at[b], sem_g).wait()
                pltpu.async_copy(
                    y_hbm.at[idxs.at[nb]], rows_v.at[nb], sem_g)
                pltpu.sync_copy(rows_v.at[b], acc.at[idxd.at[b]], add=True)
                jn = lax.rem(j + 2, per_tile)
                pltpu.async_copy(src_hbm.at[base_s + jn], idxs.at[b], sem_i)
                pltpu.async_copy(dst_hbm.at[base_d + jn], idxd.at[b], sem_i)
            return carry

        lax.fori_loop(0, per_tile // 2, body, 0)
        pltpu.make_async_copy(y_hbm.at[idxs.at[0]], rows_v.at[0], sem_g).wait()
        pltpu.make_async_copy(src_hbm.at[base_s], idxs.at[0], sem_i).wait()
        pltpu.make_async_copy(dst_hbm.at[base_d], idxd.at[0], sem_i).wait()
        plsc.subcore_barrier()
        pltpu.sync_copy(
            acc.at[pl.ds(s * rows_t, rows_t)],
            out_hbm.at[c, pl.ds(s * rows_t, rows_t)],
        )

    return agg_kernel(y_packed, src2d, dstp2d, zeros_wide)


def _dinv_of(degp_blk):
    deg = degp_blk[0, :, 0] + degp_blk[1, :, 0] + 1.0
    return lax.rsqrt(deg)


def _tc_prep1(x, W1, degp, r):
    n, d = x.shape
    h = W1.shape[1]
    hh = h // 2
    grid = (n // r,)

    def body(x_ref, w_ref, degp_ref, y1_ref):
        dinv = _dinv_of(degp_ref)
        xw = jnp.dot(x_ref[...], w_ref[...],
                     preferred_element_type=jnp.float32)
        y = xw * dinv[:, None]
        y1_ref[0] = y[:, :hh]
        y1_ref[1] = y[:, hh:]

    return pl.pallas_call(
        body,
        grid=grid,
        in_specs=[
            pl.BlockSpec((r, d), lambda i: (i, 0)),
            pl.BlockSpec((d, h), lambda i: (0, 0)),
            pl.BlockSpec((2, r, 16), lambda i: (0, i, 0)),
        ],
        out_specs=pl.BlockSpec((2, r, hh), lambda i: (0, i, 0)),
        out_shape=jax.ShapeDtypeStruct((2, n, hh), jnp.float32),
    )(x, W1, degp)


def _tc_prep2(agg1, y1, degp, b1, W2p, r):
    _, n, hh = agg1.shape
    h = 2 * hh
    cp = W2p.shape[1]
    grid = (n // r,)

    def body(agg_ref, y1_ref, degp_ref, b1_ref, w2_ref, y2_ref):
        dinv = _dinv_of(degp_ref)
        agg = jnp.concatenate([agg_ref[0], agg_ref[1]], axis=-1)
        y1b = jnp.concatenate([y1_ref[0], y1_ref[1]], axis=-1)
        hact = jnp.maximum(dinv[:, None] * (agg + y1b) + b1_ref[...], 0.0)
        y2 = jnp.dot(hact, w2_ref[...],
                     preferred_element_type=jnp.float32)
        y2_ref[...] = y2 * dinv[:, None]

    return pl.pallas_call(
        body,
        grid=grid,
        in_specs=[
            pl.BlockSpec((2, r, hh), lambda i: (0, i, 0)),
            pl.BlockSpec((2, r, hh), lambda i: (0, i, 0)),
            pl.BlockSpec((2, r, 16), lambda i: (0, i, 0)),
            pl.BlockSpec((1, h), lambda i: (0, 0)),
            pl.BlockSpec((h, cp), lambda i: (0, 0)),
        ],
        out_specs=pl.BlockSpec((r, cp), lambda i: (i, 0)),
        out_shape=jax.ShapeDtypeStruct((n, cp), jnp.float32),
    )(agg1, y1, degp, b1, W2p)


def _tc_final(agg2, y2, degp, b2p, r):
    _, npad, w = agg2.shape
    n = y2.shape[0]
    grid = (n // r,)

    def body(agg_ref, y2_ref, degp_ref, b2_ref, o_ref):
        dinv = _dinv_of(degp_ref)
        full = (dinv[:, None]
                * (agg_ref[0] + agg_ref[1] + y2_ref[...]))
        o_ref[...] = full[:, :16] + b2_ref[...]

    return pl.pallas_call(
        body,
        grid=grid,
        in_specs=[
            pl.BlockSpec((2, r, w), lambda i: (0, i, 0)),
            pl.BlockSpec((r, w), lambda i: (i, 0)),
            pl.BlockSpec((2, r, 16), lambda i: (0, i, 0)),
            pl.BlockSpec((1, 16), lambda i: (0, 0)),
        ],
        out_specs=pl.BlockSpec((r, 16), lambda i: (i, 0)),
        out_shape=jax.ShapeDtypeStruct((n, 16), jnp.float32),
    )(agg2, y2, degp, b2p)


def kernel(x, edge_index, W1, b1, W2, b2):
    n, d = x.shape
    h = W1.shape[1]
    c = W2.shape[1]
    e = edge_index.shape[1]
    hh = h // 2

    blk = _K * _NTILES
    ep = ((e + blk - 1) // blk) * blk
    p = ep - e
    pad = jnp.arange(p, dtype=jnp.int32) % _GROWS
    srcp = jnp.concatenate([edge_index[0], pad])
    dstp = jnp.concatenate([edge_index[1], pad + n])
    srcpp = jnp.concatenate([srcp, srcp + n])

    n_pad = ((n + _GROWS + 1023) // 1024) * 1024
    zeros_wide = jnp.zeros((n_pad, hh), jnp.float32)
    W2p = jnp.zeros((h, hh), jnp.float32).at[:, :c].set(W2)
    b1r = b1.reshape(1, h)
    b2p = jnp.zeros((1, 16), jnp.float32).at[0, :c].set(b2)

    r = 2000

    srcp2d = srcp.reshape(-1, _K)
    dstp2d = dstp.reshape(-1, _K)
    srcpp2d = srcpp.reshape(-1, _K)

    degp = _sc_deg(dstp2d, n_pad).reshape(2, n_pad, 16)
    y1 = _tc_prep1(x, W1, degp, r)
    y1_packed = y1.reshape(2 * n, hh)
    agg1 = _sc_agg(y1_packed, srcpp2d, dstp2d, zeros_wide, n_pad, hh, False)
    y2 = _tc_prep2(agg1, y1, degp, b1r, W2p, r)
    agg2 = _sc_agg(y2, srcp2d, dstp2d, zeros_wide, n_pad, hh, True)
    out = _tc_final(agg2, y2, degp, b2p, r)
    return out[:n, :c]
